# Initial kernel scaffold; baseline (speedup 1.0000x reference)
import jax
import jax.numpy as jnp
from jax.experimental import pallas as pl


def kernel(pred, target, ptv_mask, oar_mask_bladder, oar_mask_rectum):
    raise NotImplementedError("write your pallas kernel here")


# speedup vs baseline: 3.9542x; 3.9542x over previous
import jax
import jax.numpy as jnp
from jax.experimental import pallas as pl

_DOSE_MAX = 52.0
_QS = (0.99, 0.95, 0.01)
_B, _D, _H, _W = 2, 128, 128, 128
_SIZE = _D * _H * _W
_S = _SIZE // 128
_L = 128
_CH = 1024
_NC = _S // _CH
_I32_MAX = 2147483647
_KEY_NEG_INF = -2139095041
_KEY_NEG_ZERO = -1
_KEY_POS_INF = 2130706432


def _key_to_f32(k):
    b = jnp.where(k < 0, k ^ _I32_MAX, k)
    return jax.lax.bitcast_convert_type(b, jnp.float32)


def _isum(x):
    return jnp.sum(x.astype(jnp.int32))


def _dvh_kernel(pred_ref, tgt_ref, ptv_ref, bl_ref, re_ref, out_ref):
    outs = []
    for b in range(_B):

        def chunk(ref, c):
            return ref[b, pl.ds(c * _CH, _CH), :]

        def stat_body(c, acc):
            (n_i, cnp, cng, cb, cr,
             pbmax, gbmax, pbsum, gbsum,
             prmax, grmax, prsum, grsum) = acc
            x = chunk(pred_ref, c)
            g = chunk(tgt_ref, c)
            pm = chunk(ptv_ref, c) != 0
            bm = chunk(bl_ref, c) != 0
            rm = chunk(re_ref, c) != 0
            n_i += _isum(pm)
            cnp += _isum(jnp.logical_and(pm, x < 0.0))
            cng += _isum(jnp.logical_and(pm, g < 0.0))
            cb += _isum(bm)
            cr += _isum(rm)
            pbmax = jnp.maximum(pbmax, jnp.max(jnp.where(bm, x, -jnp.inf)))
            gbmax = jnp.maximum(gbmax, jnp.max(jnp.where(bm, g, -jnp.inf)))
            prmax = jnp.maximum(prmax, jnp.max(jnp.where(rm, x, -jnp.inf)))
            grmax = jnp.maximum(grmax, jnp.max(jnp.where(rm, g, -jnp.inf)))
            pbsum += jnp.sum(jnp.where(bm, x, 0.0))
            gbsum += jnp.sum(jnp.where(bm, g, 0.0))
            prsum += jnp.sum(jnp.where(rm, x, 0.0))
            grsum += jnp.sum(jnp.where(rm, g, 0.0))
            return (n_i, cnp, cng, cb, cr,
                    pbmax, gbmax, pbsum, gbsum,
                    prmax, grmax, prsum, grsum)

        zi = jnp.int32(0)
        zf = jnp.float32(0.0)
        ninf = jnp.float32(-jnp.inf)
        (n_i, cnp, cng, cb, cr,
         pbmax, gbmax, pbsum, gbsum,
         prmax, grmax, prsum, grsum) = jax.lax.fori_loop(
            0, _NC, stat_body,
            (zi, zi, zi, zi, zi, ninf, ninf, zf, zf, ninf, ninf, zf, zf))

        n = n_i.astype(jnp.float32)
        ptv_has = n_i > 0

        ranks, weights = [], []
        for q in _QS:
            pos = jnp.float32(q) * (n - 1.0)
            low = jnp.floor(pos)
            high = jnp.ceil(pos)
            hw = pos - low
            lw = 1.0 - hw
            li = jnp.clip(low, 0.0, float(_SIZE - 1)).astype(jnp.int32)
            hi = jnp.clip(high, 0.0, float(_SIZE - 1)).astype(jnp.int32)
            ranks.extend([li, hi])
            weights.append((lw, hw))

        def order_stats6(ref, cnt_neg):
            los, his = [], []
            for r in ranks:
                neg = (r + 1) <= cnt_neg
                los.append(jnp.where(neg, jnp.int32(_KEY_NEG_INF),
                                     jnp.int32(0)))
                his.append(jnp.where(neg, jnp.int32(_KEY_NEG_ZERO),
                                     jnp.int32(_KEY_POS_INF)))

            def sbody(_, st):
                cur_lo, cur_hi = st
                mids = []
                tmids = []
                for j in range(6):
                    mid = cur_lo[j] + jax.lax.shift_right_logical(
                        cur_hi[j] - cur_lo[j], 1)
                    mids.append(mid)
                    tmids.append(_key_to_f32(mid))

                def cbody(c, csums):
                    x = chunk(ref, c)
                    pm = chunk(ptv_ref, c) != 0
                    return tuple(
                        csums[j] + _isum(jnp.logical_and(pm, x <= tmids[j]))
                        for j in range(6))

                cnts = jax.lax.fori_loop(0, _NC, cbody, (zi,) * 6)
                nl, nh = [], []
                for j in range(6):
                    ge = cnts[j] >= (ranks[j] + 1)
                    nh.append(jnp.where(ge, mids[j], cur_hi[j]))
                    nl.append(jnp.where(ge, cur_lo[j], mids[j] + 1))
                return (tuple(nl), tuple(nh))

            _, his = jax.lax.fori_loop(0, 31, sbody, (tuple(los), tuple(his)))
            return [_key_to_f32(h) for h in his]

        vp = order_stats6(pred_ref, cnp)
        vg = order_stats6(tgt_ref, cng)

        loss = jnp.float32(0.0)
        for qi in range(3):
            lw, hw = weights[qi]
            pv = (vp[2 * qi] * _DOSE_MAX) * lw + (vp[2 * qi + 1] * _DOSE_MAX) * hw
            gv = (vg[2 * qi] * _DOSE_MAX) * lw + (vg[2 * qi + 1] * _DOSE_MAX) * hw
            loss += jnp.where(ptv_has, jnp.abs(pv - gv), 0.0)

        valid = ptv_has
        for cnt, pmax, gmax, psum, gsum in (
                (cb, pbmax, gbmax, pbsum, gbsum),
                (cr, prmax, grmax, prsum, grsum)):
            has = cnt > 0
            valid = jnp.logical_or(valid, has)
            cf = cnt.astype(jnp.float32)
            loss += jnp.where(
                has, jnp.abs(pmax * _DOSE_MAX - gmax * _DOSE_MAX), 0.0)
            loss += jnp.where(
                has,
                jnp.abs(psum * _DOSE_MAX / cf - gsum * _DOSE_MAX / cf), 0.0)

        outs.append((loss, valid.astype(jnp.float32)))

    lane = jax.lax.broadcasted_iota(jnp.int32, (1, _L), 1)
    row = jnp.where(lane == 0, outs[0][0],
                    jnp.where(lane == 1, outs[0][1],
                              jnp.where(lane == 2, outs[1][0],
                                        jnp.where(lane == 3, outs[1][1],
                                                  0.0))))
    out_ref[...] = row


def kernel(pred, target, ptv_mask, oar_mask_bladder, oar_mask_rectum):
    pred3 = pred.astype(jnp.float32).reshape(_B, _S, _L)
    tgt3 = target.astype(jnp.float32).reshape(_B, _S, _L)
    ptv3 = ptv_mask.astype(jnp.int8).reshape(_B, _S, _L)
    bl3 = oar_mask_bladder.astype(jnp.int8).reshape(_B, _S, _L)
    re3 = oar_mask_rectum.astype(jnp.int8).reshape(_B, _S, _L)

    out = pl.pallas_call(
        _dvh_kernel,
        out_shape=jax.ShapeDtypeStruct((1, _L), jnp.float32),
    )(pred3, tgt3, ptv3, bl3, re3)

    losses = jnp.stack([out[0, 0], out[0, 2]])
    valid = jnp.stack([out[0, 1], out[0, 3]])
    num_valid = valid.sum()
    return jnp.where(
        num_valid > 0,
        (losses * valid).sum() / num_valid,
        jnp.asarray(0.0, dtype=jnp.float32),
    )


# speedup vs baseline: 5.1459x; 1.3014x over previous
import jax
import jax.numpy as jnp
from jax.experimental import pallas as pl

_DOSE_MAX = 52.0
_QS = (0.99, 0.95, 0.01)
_B, _D, _H, _W = 2, 128, 128, 128
_SIZE = _D * _H * _W
_S = _SIZE // 128
_L = 128
_CH = 1024
_NC = _S // _CH
_I32_MAX = 2147483647


def _f32_to_key(v):
    b = jax.lax.bitcast_convert_type(v, jnp.int32)
    return jnp.where(b < 0, b ^ _I32_MAX, b)


def _key_to_f32(k):
    b = jnp.where(k < 0, k ^ _I32_MAX, k)
    return jax.lax.bitcast_convert_type(b, jnp.float32)


def _isum(x):
    return jnp.sum(x.astype(jnp.int32))


def _dvh_kernel(pred_ref, tgt_ref, ptv_ref, bl_ref, re_ref, out_ref):
    outs = []
    for b in range(_B):

        def chunk(ref, c):
            return ref[b, pl.ds(c * _CH, _CH), :]

        def stat_body(c, acc):
            (n_i, cb, cr,
             pmn, pmx, gmn, gmx,
             pbmax, gbmax, pbsum, gbsum,
             prmax, grmax, prsum, grsum) = acc
            x = chunk(pred_ref, c)
            g = chunk(tgt_ref, c)
            pm = chunk(ptv_ref, c) != 0
            bm = chunk(bl_ref, c) != 0
            rm = chunk(re_ref, c) != 0
            n_i += _isum(pm)
            cb += _isum(bm)
            cr += _isum(rm)
            pmn = jnp.minimum(pmn, jnp.min(jnp.where(pm, x, jnp.inf)))
            pmx = jnp.maximum(pmx, jnp.max(jnp.where(pm, x, -jnp.inf)))
            gmn = jnp.minimum(gmn, jnp.min(jnp.where(pm, g, jnp.inf)))
            gmx = jnp.maximum(gmx, jnp.max(jnp.where(pm, g, -jnp.inf)))
            pbmax = jnp.maximum(pbmax, jnp.max(jnp.where(bm, x, -jnp.inf)))
            gbmax = jnp.maximum(gbmax, jnp.max(jnp.where(bm, g, -jnp.inf)))
            prmax = jnp.maximum(prmax, jnp.max(jnp.where(rm, x, -jnp.inf)))
            grmax = jnp.maximum(grmax, jnp.max(jnp.where(rm, g, -jnp.inf)))
            pbsum += jnp.sum(jnp.where(bm, x, 0.0))
            gbsum += jnp.sum(jnp.where(bm, g, 0.0))
            prsum += jnp.sum(jnp.where(rm, x, 0.0))
            grsum += jnp.sum(jnp.where(rm, g, 0.0))
            return (n_i, cb, cr,
                    pmn, pmx, gmn, gmx,
                    pbmax, gbmax, pbsum, gbsum,
                    prmax, grmax, prsum, grsum)

        zi = jnp.int32(0)
        zf = jnp.float32(0.0)
        ninf = jnp.float32(-jnp.inf)
        pinf = jnp.float32(jnp.inf)
        (n_i, cb, cr,
         pmn, pmx, gmn, gmx,
         pbmax, gbmax, pbsum, gbsum,
         prmax, grmax, prsum, grsum) = jax.lax.fori_loop(
            0, _NC, stat_body,
            (zi, zi, zi, pinf, ninf, pinf, ninf,
             ninf, ninf, zf, zf, ninf, ninf, zf, zf))

        n = n_i.astype(jnp.float32)
        ptv_has = n_i > 0

        ranks, weights = [], []
        for q in _QS:
            pos = jnp.float32(q) * (n - 1.0)
            low = jnp.floor(pos)
            high = jnp.ceil(pos)
            hw = pos - low
            lw = 1.0 - hw
            li = jnp.clip(low, 0.0, float(_SIZE - 1)).astype(jnp.int32)
            hi = jnp.clip(high, 0.0, float(_SIZE - 1)).astype(jnp.int32)
            ranks.extend([li, hi])
            weights.append((lw, hw))

        def order_stats6(ref, mn, mx):
            k_lo0 = _f32_to_key(mn) - 1
            k_hi0 = _f32_to_key(mx)
            los = [k_lo0] * 6
            his = [k_hi0] * 6
            clos = [zi] * 6
            chis = [n_i] * 6

            def cond(st):
                _, cur_lo, cur_hi, _, _ = st
                unconv = (cur_lo[0] < cur_hi[0] - 1)
                for j in range(1, 6):
                    unconv = jnp.logical_or(unconv,
                                            cur_lo[j] < cur_hi[j] - 1)
                return unconv

            def body(st):
                it, cur_lo, cur_hi, cur_cl, cur_ch = st
                kms, kis, ts = [], [], []
                for j in range(6):
                    lo, hi = cur_lo[j], cur_hi[j]
                    cl, ch = cur_cl[j], cur_ch[j]
                    km = (lo >> 1) + (hi >> 1) + (lo & hi & 1)
                    vlo = _key_to_f32(lo)
                    vhi = _key_to_f32(hi)
                    frac = (ranks[j] + 1 - cl).astype(jnp.float32) / \
                        jnp.maximum((ch - cl).astype(jnp.float32), 1.0)
                    t = vlo + (vhi - vlo) * frac
                    ki = jnp.clip(_f32_to_key(t), lo + 1, hi - 1)
                    kms.append(km)
                    kis.append(ki)
                    ts.append(_key_to_f32(km))
                    ts.append(_key_to_f32(ki))

                def cbody(c, csums):
                    x = chunk(ref, c)
                    pm = chunk(ptv_ref, c) != 0
                    return tuple(
                        csums[t] + _isum(jnp.logical_and(pm, x <= ts[t]))
                        for t in range(12))

                cnts = jax.lax.fori_loop(0, _NC, cbody, (zi,) * 12)

                def upd(lo, hi, cl, ch, k, c, r):
                    ge = c >= (r + 1)
                    bh = jnp.logical_and(ge, k < hi)
                    hi = jnp.where(bh, k, hi)
                    ch = jnp.where(bh, c, ch)
                    bl_ = jnp.logical_and(jnp.logical_not(ge), k > lo)
                    lo = jnp.where(bl_, k, lo)
                    cl = jnp.where(bl_, c, cl)
                    return lo, hi, cl, ch

                nl, nh, ncl, nch = [], [], [], []
                for j in range(6):
                    lo, hi = cur_lo[j], cur_hi[j]
                    cl, ch = cur_cl[j], cur_ch[j]
                    lo, hi, cl, ch = upd(lo, hi, cl, ch,
                                         kms[j], cnts[2 * j], ranks[j])
                    lo, hi, cl, ch = upd(lo, hi, cl, ch,
                                         kis[j], cnts[2 * j + 1], ranks[j])
                    nl.append(lo)
                    nh.append(hi)
                    ncl.append(cl)
                    nch.append(ch)
                return (it + 1, tuple(nl), tuple(nh), tuple(ncl), tuple(nch))

            _, _, his, _, _ = jax.lax.while_loop(
                cond, body,
                (zi, tuple(los), tuple(his), tuple(clos), tuple(chis)))
            return [_key_to_f32(h) for h in his]

        vp = order_stats6(pred_ref, pmn, pmx)
        vg = order_stats6(tgt_ref, gmn, gmx)

        loss = jnp.float32(0.0)
        for qi in range(3):
            lw, hw = weights[qi]
            pv = (vp[2 * qi] * _DOSE_MAX) * lw + (vp[2 * qi + 1] * _DOSE_MAX) * hw
            gv = (vg[2 * qi] * _DOSE_MAX) * lw + (vg[2 * qi + 1] * _DOSE_MAX) * hw
            loss += jnp.where(ptv_has, jnp.abs(pv - gv), 0.0)

        valid = ptv_has
        for cnt, pmax, gmax, psum, gsum in (
                (cb, pbmax, gbmax, pbsum, gbsum),
                (cr, prmax, grmax, prsum, grsum)):
            has = cnt > 0
            valid = jnp.logical_or(valid, has)
            cf = cnt.astype(jnp.float32)
            loss += jnp.where(
                has, jnp.abs(pmax * _DOSE_MAX - gmax * _DOSE_MAX), 0.0)
            loss += jnp.where(
                has,
                jnp.abs(psum * _DOSE_MAX / cf - gsum * _DOSE_MAX / cf), 0.0)

        outs.append((loss, valid.astype(jnp.float32)))

    lane = jax.lax.broadcasted_iota(jnp.int32, (1, _L), 1)
    row = jnp.where(lane == 0, outs[0][0],
                    jnp.where(lane == 1, outs[0][1],
                              jnp.where(lane == 2, outs[1][0],
                                        jnp.where(lane == 3, outs[1][1],
                                                  0.0))))
    out_ref[...] = row


def kernel(pred, target, ptv_mask, oar_mask_bladder, oar_mask_rectum):
    pred3 = pred.astype(jnp.float32).reshape(_B, _S, _L)
    tgt3 = target.astype(jnp.float32).reshape(_B, _S, _L)
    ptv3 = ptv_mask.astype(jnp.int8).reshape(_B, _S, _L)
    bl3 = oar_mask_bladder.astype(jnp.int8).reshape(_B, _S, _L)
    re3 = oar_mask_rectum.astype(jnp.int8).reshape(_B, _S, _L)

    out = pl.pallas_call(
        _dvh_kernel,
        out_shape=jax.ShapeDtypeStruct((1, _L), jnp.float32),
    )(pred3, tgt3, ptv3, bl3, re3)

    losses = jnp.stack([out[0, 0], out[0, 2]])
    valid = jnp.stack([out[0, 1], out[0, 3]])
    num_valid = valid.sum()
    return jnp.where(
        num_valid > 0,
        (losses * valid).sum() / num_valid,
        jnp.asarray(0.0, dtype=jnp.float32),
    )


# speedup vs baseline: 23.6417x; 4.5943x over previous
import jax
import jax.numpy as jnp
from jax.experimental import pallas as pl
from jax.experimental.pallas import tpu as pltpu

_DOSE_MAX = 52.0
_QS = (0.99, 0.95, 0.01)
_B, _D, _H, _W = 2, 128, 128, 128
_SIZE = _D * _H * _W
_S = _SIZE // 128
_L = 128
_CH = 1024
_NC = _S // _CH
_I32_MAX = 2147483647


def _f32_to_key(v):
    b = jax.lax.bitcast_convert_type(v, jnp.int32)
    return jnp.where(b < 0, b ^ _I32_MAX, b)


def _key_to_f32(k):
    b = jnp.where(k < 0, k ^ _I32_MAX, k)
    return jax.lax.bitcast_convert_type(b, jnp.float32)


def _isum(x):
    return jnp.sum(x.astype(jnp.int32))


def _dvh_kernel(pred_ref, tgt_ref, msk_ref, out_ref, pxm_ref, gxm_ref):
    nan = jnp.float32(jnp.nan)
    outs = []
    for b in range(_B):

        def chunk(ref, c):
            return ref[b, pl.ds(c * _CH, _CH), :]

        def schunk(ref, c):
            return ref[pl.ds(c * _CH, _CH), :]

        def stat_body(c, acc):
            (n_i, cb, cr,
             pmn, pmx, gmn, gmx,
             pbmax, gbmax, pbsum, gbsum,
             prmax, grmax, prsum, grsum) = acc
            x = chunk(pred_ref, c)
            g = chunk(tgt_ref, c)
            mk = chunk(msk_ref, c)
            pm = (mk & 1) != 0
            bm = (mk & 2) != 0
            rm = (mk & 4) != 0
            pxm_ref[pl.ds(c * _CH, _CH), :] = jnp.where(pm, x, nan)
            gxm_ref[pl.ds(c * _CH, _CH), :] = jnp.where(pm, g, nan)
            n_i += _isum(pm)
            cb += _isum(bm)
            cr += _isum(rm)
            pmn = jnp.minimum(pmn, jnp.min(jnp.where(pm, x, jnp.inf)))
            pmx = jnp.maximum(pmx, jnp.max(jnp.where(pm, x, -jnp.inf)))
            gmn = jnp.minimum(gmn, jnp.min(jnp.where(pm, g, jnp.inf)))
            gmx = jnp.maximum(gmx, jnp.max(jnp.where(pm, g, -jnp.inf)))
            pbmax = jnp.maximum(pbmax, jnp.max(jnp.where(bm, x, -jnp.inf)))
            gbmax = jnp.maximum(gbmax, jnp.max(jnp.where(bm, g, -jnp.inf)))
            prmax = jnp.maximum(prmax, jnp.max(jnp.where(rm, x, -jnp.inf)))
            grmax = jnp.maximum(grmax, jnp.max(jnp.where(rm, g, -jnp.inf)))
            pbsum += jnp.sum(jnp.where(bm, x, 0.0))
            gbsum += jnp.sum(jnp.where(bm, g, 0.0))
            prsum += jnp.sum(jnp.where(rm, x, 0.0))
            grsum += jnp.sum(jnp.where(rm, g, 0.0))
            return (n_i, cb, cr,
                    pmn, pmx, gmn, gmx,
                    pbmax, gbmax, pbsum, gbsum,
                    prmax, grmax, prsum, grsum)

        zi = jnp.int32(0)
        zf = jnp.float32(0.0)
        ninf = jnp.float32(-jnp.inf)
        pinf = jnp.float32(jnp.inf)
        (n_i, cb, cr,
         pmn, pmx, gmn, gmx,
         pbmax, gbmax, pbsum, gbsum,
         prmax, grmax, prsum, grsum) = jax.lax.fori_loop(
            0, _NC, stat_body,
            (zi, zi, zi, pinf, ninf, pinf, ninf,
             ninf, ninf, zf, zf, ninf, ninf, zf, zf))

        n = n_i.astype(jnp.float32)
        ptv_has = n_i > 0

        ranks, hi_ranks, weights = [], [], []
        for q in _QS:
            pos = jnp.float32(q) * (n - 1.0)
            low = jnp.floor(pos)
            high = jnp.ceil(pos)
            hw = pos - low
            lw = 1.0 - hw
            li = jnp.clip(low, 0.0, float(_SIZE - 1)).astype(jnp.int32)
            hi = jnp.clip(high, 0.0, float(_SIZE - 1)).astype(jnp.int32)
            ranks.append(li)
            hi_ranks.append(hi)
            weights.append((lw, hw))

        def order_stats(xm_ref, mn, mx):
            k_lo0 = _f32_to_key(mn) - 1
            k_hi0 = _f32_to_key(mx)
            los = [k_lo0] * 3
            his = [k_hi0] * 3
            clos = [zi] * 3
            chis = [n_i] * 3

            def cond(st):
                cur_lo, cur_hi = st[0], st[1]
                unconv = cur_lo[0] < cur_hi[0] - 1
                for j in range(1, 3):
                    unconv = jnp.logical_or(unconv,
                                            cur_lo[j] < cur_hi[j] - 1)
                return unconv

            def body(st):
                cur_lo, cur_hi, cur_cl, cur_ch = st
                kms, kis, ts = [], [], []
                for j in range(3):
                    lo, hi = cur_lo[j], cur_hi[j]
                    cl, ch = cur_cl[j], cur_ch[j]
                    km = (lo >> 1) + (hi >> 1) + (lo & hi & 1)
                    vlo = _key_to_f32(lo)
                    vhi = _key_to_f32(hi)
                    frac = (ranks[j] + 1 - cl).astype(jnp.float32) / \
                        jnp.maximum((ch - cl).astype(jnp.float32), 1.0)
                    t = vlo + (vhi - vlo) * frac
                    ki = jnp.clip(_f32_to_key(t), lo + 1, hi - 1)
                    kms.append(km)
                    kis.append(ki)
                    ts.append(_key_to_f32(km))
                    ts.append(_key_to_f32(ki))

                def cbody(c, csums):
                    xm = schunk(xm_ref, c)
                    return tuple(
                        csums[t] + _isum(xm <= ts[t]) for t in range(6))

                cnts = jax.lax.fori_loop(0, _NC, cbody, (zi,) * 6)

                def upd(lo, hi, cl, ch, k, c, r):
                    ge = c >= (r + 1)
                    bh = jnp.logical_and(ge, k < hi)
                    hi = jnp.where(bh, k, hi)
                    ch = jnp.where(bh, c, ch)
                    bl_ = jnp.logical_and(jnp.logical_not(ge), k > lo)
                    lo = jnp.where(bl_, k, lo)
                    cl = jnp.where(bl_, c, cl)
                    return lo, hi, cl, ch

                nl, nh, ncl, nch = [], [], [], []
                for j in range(3):
                    lo, hi = cur_lo[j], cur_hi[j]
                    cl, ch = cur_cl[j], cur_ch[j]
                    lo, hi, cl, ch = upd(lo, hi, cl, ch,
                                         kms[j], cnts[2 * j], ranks[j])
                    lo, hi, cl, ch = upd(lo, hi, cl, ch,
                                         kis[j], cnts[2 * j + 1], ranks[j])
                    nl.append(lo)
                    nh.append(hi)
                    ncl.append(cl)
                    nch.append(ch)
                return (tuple(nl), tuple(nh), tuple(ncl), tuple(nch))

            _, his, _, chis = jax.lax.while_loop(
                cond, body,
                (tuple(los), tuple(his), tuple(clos), tuple(chis)))
            return [_key_to_f32(h) for h in his], chis

        vp, cp = order_stats(pxm_ref, pmn, pmx)
        vg, cg = order_stats(gxm_ref, gmn, gmx)

        def next_body(c, acc):
            xm = schunk(pxm_ref, c)
            gm = schunk(gxm_ref, c)
            new = []
            for j in range(3):
                new.append(jnp.minimum(
                    acc[j], jnp.min(jnp.where(xm > vp[j], xm, jnp.inf))))
            for j in range(3):
                new.append(jnp.minimum(
                    acc[3 + j], jnp.min(jnp.where(gm > vg[j], gm, jnp.inf))))
            return tuple(new)

        nxt = jax.lax.fori_loop(0, _NC, next_body, (pinf,) * 6)

        loss = jnp.float32(0.0)
        for qi in range(3):
            lw, hw = weights[qi]
            same = hi_ranks[qi] == ranks[qi]
            pv_hi = jnp.where(same, vp[qi],
                              jnp.where(cp[qi] >= ranks[qi] + 2,
                                        vp[qi], nxt[qi]))
            gv_hi = jnp.where(same, vg[qi],
                              jnp.where(cg[qi] >= ranks[qi] + 2,
                                        vg[qi], nxt[3 + qi]))
            pv = (vp[qi] * _DOSE_MAX) * lw + (pv_hi * _DOSE_MAX) * hw
            gv = (vg[qi] * _DOSE_MAX) * lw + (gv_hi * _DOSE_MAX) * hw
            loss += jnp.where(ptv_has, jnp.abs(pv - gv), 0.0)

        valid = ptv_has
        for cnt, pmax, gmax, psum, gsum in (
                (cb, pbmax, gbmax, pbsum, gbsum),
                (cr, prmax, grmax, prsum, grsum)):
            has = cnt > 0
            valid = jnp.logical_or(valid, has)
            cf = cnt.astype(jnp.float32)
            loss += jnp.where(
                has, jnp.abs(pmax * _DOSE_MAX - gmax * _DOSE_MAX), 0.0)
            loss += jnp.where(
                has,
                jnp.abs(psum * _DOSE_MAX / cf - gsum * _DOSE_MAX / cf), 0.0)

        outs.append((loss, valid.astype(jnp.float32)))

    lane = jax.lax.broadcasted_iota(jnp.int32, (1, _L), 1)
    row = jnp.where(lane == 0, outs[0][0],
                    jnp.where(lane == 1, outs[0][1],
                              jnp.where(lane == 2, outs[1][0],
                                        jnp.where(lane == 3, outs[1][1],
                                                  0.0))))
    out_ref[...] = row


def kernel(pred, target, ptv_mask, oar_mask_bladder, oar_mask_rectum):
    pred3 = pred.astype(jnp.float32).reshape(_B, _S, _L)
    tgt3 = target.astype(jnp.float32).reshape(_B, _S, _L)
    msk3 = (ptv_mask.astype(jnp.int8)
            + oar_mask_bladder.astype(jnp.int8) * 2
            + oar_mask_rectum.astype(jnp.int8) * 4).reshape(_B, _S, _L)

    out = pl.pallas_call(
        _dvh_kernel,
        out_shape=jax.ShapeDtypeStruct((1, _L), jnp.float32),
        scratch_shapes=[pltpu.VMEM((_S, _L), jnp.float32),
                        pltpu.VMEM((_S, _L), jnp.float32)],
    )(pred3, tgt3, msk3)

    losses = jnp.stack([out[0, 0], out[0, 2]])
    valid = jnp.stack([out[0, 1], out[0, 3]])
    num_valid = valid.sum()
    return jnp.where(
        num_valid > 0,
        (losses * valid).sum() / num_valid,
        jnp.asarray(0.0, dtype=jnp.float32),
    )


# speedup vs baseline: 33.7941x; 1.4294x over previous
import jax
import jax.numpy as jnp
from jax.experimental import pallas as pl
from jax.experimental.pallas import tpu as pltpu

_DOSE_MAX = 52.0
_QS = (0.99, 0.95, 0.01)
_B, _D, _H, _W = 2, 128, 128, 128
_SIZE = _D * _H * _W
_S = _SIZE // 128
_L = 128
_CH = 1024
_NC = _S // _CH
_I32_MAX = 2147483647
_CGAP = 3
_NCHAIN = _CGAP + 1


def _f32_to_key(v):
    v = v + jnp.float32(0.0)
    b = jax.lax.bitcast_convert_type(v, jnp.int32)
    return jnp.where(b < 0, b ^ _I32_MAX, b)


def _key_to_f32(k):
    b = jnp.where(k < 0, k ^ _I32_MAX, k)
    return jax.lax.bitcast_convert_type(b, jnp.float32)


def _isum(x):
    return jnp.sum(x.astype(jnp.int32))


def _dvh_kernel(pred_ref, tgt_ref, msk_ref, out_ref, pxm_ref, gxm_ref):
    nan = jnp.float32(jnp.nan)
    outs = []
    for b in range(_B):

        def chunk(ref, c):
            return ref[b, pl.ds(c * _CH, _CH), :]

        def schunk(ref, c):
            return ref[pl.ds(c * _CH, _CH), :]

        def stat_body(c, acc):
            (n_i, cb, cr,
             pmn, pmx, gmn, gmx,
             pbmax, gbmax, pbsum, gbsum,
             prmax, grmax, prsum, grsum) = acc
            x = chunk(pred_ref, c)
            g = chunk(tgt_ref, c)
            mk = chunk(msk_ref, c)
            pm = (mk & 1) != 0
            bm = (mk & 2) != 0
            rm = (mk & 4) != 0
            pxm_ref[pl.ds(c * _CH, _CH), :] = jnp.where(pm, x, nan)
            gxm_ref[pl.ds(c * _CH, _CH), :] = jnp.where(pm, g, nan)
            n_i += _isum(pm)
            cb += _isum(bm)
            cr += _isum(rm)
            pmn = jnp.minimum(pmn, jnp.min(jnp.where(pm, x, jnp.inf)))
            pmx = jnp.maximum(pmx, jnp.max(jnp.where(pm, x, -jnp.inf)))
            gmn = jnp.minimum(gmn, jnp.min(jnp.where(pm, g, jnp.inf)))
            gmx = jnp.maximum(gmx, jnp.max(jnp.where(pm, g, -jnp.inf)))
            pbmax = jnp.maximum(pbmax, jnp.max(jnp.where(bm, x, -jnp.inf)))
            gbmax = jnp.maximum(gbmax, jnp.max(jnp.where(bm, g, -jnp.inf)))
            prmax = jnp.maximum(prmax, jnp.max(jnp.where(rm, x, -jnp.inf)))
            grmax = jnp.maximum(grmax, jnp.max(jnp.where(rm, g, -jnp.inf)))
            pbsum += jnp.sum(jnp.where(bm, x, 0.0))
            gbsum += jnp.sum(jnp.where(bm, g, 0.0))
            prsum += jnp.sum(jnp.where(rm, x, 0.0))
            grsum += jnp.sum(jnp.where(rm, g, 0.0))
            return (n_i, cb, cr,
                    pmn, pmx, gmn, gmx,
                    pbmax, gbmax, pbsum, gbsum,
                    prmax, grmax, prsum, grsum)

        zi = jnp.int32(0)
        zf = jnp.float32(0.0)
        ninf = jnp.float32(-jnp.inf)
        pinf = jnp.float32(jnp.inf)
        (n_i, cb, cr,
         pmn, pmx, gmn, gmx,
         pbmax, gbmax, pbsum, gbsum,
         prmax, grmax, prsum, grsum) = jax.lax.fori_loop(
            0, _NC, stat_body,
            (zi, zi, zi, pinf, ninf, pinf, ninf,
             ninf, ninf, zf, zf, ninf, ninf, zf, zf))

        n = n_i.astype(jnp.float32)
        ptv_has = n_i > 0

        ranks, hi_ranks, weights = [], [], []
        for q in _QS:
            pos = jnp.float32(q) * (n - 1.0)
            low = jnp.floor(pos)
            high = jnp.ceil(pos)
            hw = pos - low
            lw = 1.0 - hw
            li = jnp.clip(low, 0.0, float(_SIZE - 1)).astype(jnp.int32)
            hi = jnp.clip(high, 0.0, float(_SIZE - 1)).astype(jnp.int32)
            ranks.append(li)
            hi_ranks.append(hi)
            weights.append((lw, hw))

        def bracket3(xm_ref, mn, mx):
            k_lo0 = _f32_to_key(mn) - 1
            k_lo0 = jnp.where(_key_to_f32(k_lo0) == 0.0,
                              jnp.int32(-8388609), k_lo0)
            k_hi0 = _f32_to_key(mx)
            los = [k_lo0] * 3
            his = [k_hi0] * 3
            clos = [zi] * 3
            chis = [n_i] * 3

            def cond(st):
                cur_lo, cur_hi, cur_cl, cur_ch = st
                unconv = jnp.logical_and(cur_lo[0] < cur_hi[0] - 1,
                                         cur_ch[0] - cur_cl[0] > _CGAP)
                for j in range(1, 3):
                    unconv = jnp.logical_or(
                        unconv,
                        jnp.logical_and(cur_lo[j] < cur_hi[j] - 1,
                                        cur_ch[j] - cur_cl[j] > _CGAP))
                return unconv

            def body(st):
                cur_lo, cur_hi, cur_cl, cur_ch = st
                kms, kis, ts = [], [], []
                for j in range(3):
                    lo, hi = cur_lo[j], cur_hi[j]
                    cl, ch = cur_cl[j], cur_ch[j]
                    km = (lo >> 1) + (hi >> 1) + (lo & hi & 1)
                    vlo = _key_to_f32(lo)
                    vhi = _key_to_f32(hi)
                    frac = (ranks[j] + 1 - cl).astype(jnp.float32) / \
                        jnp.maximum((ch - cl).astype(jnp.float32), 1.0)
                    t = vlo + (vhi - vlo) * frac
                    ki = jnp.clip(_f32_to_key(t), lo + 1, hi - 1)
                    kms.append(km)
                    kis.append(ki)
                    ts.append(_key_to_f32(km))
                    ts.append(_key_to_f32(ki))

                def cbody(c, csums):
                    xm = schunk(xm_ref, c)
                    return tuple(
                        csums[t] + _isum(xm <= ts[t]) for t in range(6))

                cnts = jax.lax.fori_loop(0, _NC, cbody, (zi,) * 6)

                def upd(lo, hi, cl, ch, k, c, r):
                    ge = c >= (r + 1)
                    bh = jnp.logical_and(ge, k < hi)
                    hi = jnp.where(bh, k, hi)
                    ch = jnp.where(bh, c, ch)
                    bl_ = jnp.logical_and(jnp.logical_not(ge), k > lo)
                    lo = jnp.where(bl_, k, lo)
                    cl = jnp.where(bl_, c, cl)
                    return lo, hi, cl, ch

                nl, nh, ncl, nch = [], [], [], []
                for j in range(3):
                    lo, hi = cur_lo[j], cur_hi[j]
                    cl, ch = cur_cl[j], cur_ch[j]
                    lo, hi, cl, ch = upd(lo, hi, cl, ch,
                                         kms[j], cnts[2 * j], ranks[j])
                    lo, hi, cl, ch = upd(lo, hi, cl, ch,
                                         kis[j], cnts[2 * j + 1], ranks[j])
                    nl.append(lo)
                    nh.append(hi)
                    ncl.append(cl)
                    nch.append(ch)
                return (tuple(nl), tuple(nh), tuple(ncl), tuple(nch))

            los, _, clos, _ = jax.lax.while_loop(
                cond, body,
                (tuple(los), tuple(his), tuple(clos), tuple(chis)))
            return [_key_to_f32(l) for l in los], list(clos)

        pu, pc = bracket3(pxm_ref, pmn, pmx)
        gu, gc = bracket3(gxm_ref, gmn, gmx)

        us = [[v] for v in pu + gu]
        ds = [[] for _ in range(6)]
        for _k in range(_NCHAIN):
            prev = [us[t][-1] for t in range(6)]

            def chain_body(c, acc):
                xm = schunk(pxm_ref, c)
                gm = schunk(gxm_ref, c)
                new = []
                for t in range(6):
                    arr = xm if t < 3 else gm
                    above = arr > prev[t]
                    mn_t = jnp.minimum(
                        acc[2 * t], jnp.min(jnp.where(above, arr, jnp.inf)))
                    d_t = acc[2 * t + 1] + _isum(above)
                    new.extend([mn_t, d_t])
                return tuple(new)

            init = (pinf, zi) * 6
            res = jax.lax.fori_loop(0, _NC, chain_body, init)
            for t in range(6):
                us[t].append(res[2 * t])
                ds[t].append(res[2 * t + 1])
        cs = [[(pc + gc)[t]] + [n_i - ds[t][k] for k in range(1, _NCHAIN)]
              for t in range(6)]

        def pick(chain_u, chain_c, need):
            val = chain_u[_NCHAIN]
            for k in range(_NCHAIN - 1, -1, -1):
                val = jnp.where(chain_c[k] >= need, chain_u[k], val)
            return val

        loss = jnp.float32(0.0)
        for qi in range(3):
            lw, hw = weights[qi]
            same = hi_ranks[qi] == ranks[qi]
            pv_lo = pick(us[qi], cs[qi], ranks[qi] + 1)
            gv_lo = pick(us[3 + qi], cs[3 + qi], ranks[qi] + 1)
            pv_hi = jnp.where(same, pv_lo,
                              pick(us[qi], cs[qi], ranks[qi] + 2))
            gv_hi = jnp.where(same, gv_lo,
                              pick(us[3 + qi], cs[3 + qi], ranks[qi] + 2))
            pv = (pv_lo * _DOSE_MAX) * lw + (pv_hi * _DOSE_MAX) * hw
            gv = (gv_lo * _DOSE_MAX) * lw + (gv_hi * _DOSE_MAX) * hw
            loss += jnp.where(ptv_has, jnp.abs(pv - gv), 0.0)

        valid = ptv_has
        for cnt, pmax, gmax, psum, gsum in (
                (cb, pbmax, gbmax, pbsum, gbsum),
                (cr, prmax, grmax, prsum, grsum)):
            has = cnt > 0
            valid = jnp.logical_or(valid, has)
            cf = cnt.astype(jnp.float32)
            loss += jnp.where(
                has, jnp.abs(pmax * _DOSE_MAX - gmax * _DOSE_MAX), 0.0)
            loss += jnp.where(
                has,
                jnp.abs(psum * _DOSE_MAX / cf - gsum * _DOSE_MAX / cf), 0.0)

        outs.append((loss, valid.astype(jnp.float32)))

    lane = jax.lax.broadcasted_iota(jnp.int32, (1, _L), 1)
    row = jnp.where(lane == 0, outs[0][0],
                    jnp.where(lane == 1, outs[0][1],
                              jnp.where(lane == 2, outs[1][0],
                                        jnp.where(lane == 3, outs[1][1],
                                                  0.0))))
    out_ref[...] = row


def kernel(pred, target, ptv_mask, oar_mask_bladder, oar_mask_rectum):
    pred3 = pred.astype(jnp.float32).reshape(_B, _S, _L)
    tgt3 = target.astype(jnp.float32).reshape(_B, _S, _L)
    msk3 = (ptv_mask.astype(jnp.int8)
            + oar_mask_bladder.astype(jnp.int8) * 2
            + oar_mask_rectum.astype(jnp.int8) * 4).reshape(_B, _S, _L)

    out = pl.pallas_call(
        _dvh_kernel,
        out_shape=jax.ShapeDtypeStruct((1, _L), jnp.float32),
        scratch_shapes=[pltpu.VMEM((_S, _L), jnp.float32),
                        pltpu.VMEM((_S, _L), jnp.float32)],
    )(pred3, tgt3, msk3)

    losses = jnp.stack([out[0, 0], out[0, 2]])
    valid = jnp.stack([out[0, 1], out[0, 3]])
    num_valid = valid.sum()
    return jnp.where(
        num_valid > 0,
        (losses * valid).sum() / num_valid,
        jnp.asarray(0.0, dtype=jnp.float32),
    )
